# baseline (speedup 1.0000x reference)
import jax
import jax.numpy as jnp
from jax import lax
from jax.experimental import pallas as pl
from jax.experimental.pallas import tpu as pltpu
from jax.experimental.pallas import tpu_sc as plsc

_NUM_CORES = 2
_NUM_SUBCORES = 16
_NUM_WORKERS = _NUM_CORES * _NUM_SUBCORES
_BAND_ROWS = 8
_NUM_BANDS = 8
_NUM_SPANS = 4
_COLS_PER_SPAN = 249856
_CHUNK_COLS = 2048
_DEPTH = 3
_SLOTS = 7


def _sc_copy_body(src_hbm, dst_hbm, bufs, in_sems, out_sems):
    nchunks = _COLS_PER_SPAN // _CHUNK_COLS
    wid = lax.axis_index("s") * _NUM_CORES + lax.axis_index("c")
    band = lax.rem(wid, _NUM_BANDS)
    span = lax.div(wid, _NUM_BANDS)
    row0 = pl.multiple_of(band * _BAND_ROWS, 8)
    col0 = pl.multiple_of(span * _COLS_PER_SPAN, 128)

    def in_copy(c, slot):
        return pltpu.make_async_copy(
            src_hbm.at[pl.ds(row0, _BAND_ROWS), pl.ds(col0 + c * _CHUNK_COLS, _CHUNK_COLS)],
            bufs.at[slot],
            in_sems.at[slot],
        )

    def out_copy(c, slot):
        return pltpu.make_async_copy(
            bufs.at[slot],
            dst_hbm.at[pl.ds(row0, _BAND_ROWS), pl.ds(col0 + c * _CHUNK_COLS, _CHUNK_COLS)],
            out_sems.at[slot],
        )

    for c in range(_DEPTH):
        in_copy(c, c % _SLOTS).start()

    for i in range(nchunks):
        slot = i % _SLOTS
        in_copy(i, slot).wait()
        out_copy(i, slot).start()
        nxt = i + _DEPTH
        if nxt < nchunks:
            nslot = nxt % _SLOTS
            if nxt >= _SLOTS:
                out_copy(nxt - _SLOTS, nslot).wait()
            in_copy(nxt, nslot).start()

    for k in range(min(_SLOTS, nchunks)):
        c = nchunks - min(_SLOTS, nchunks) + k
        out_copy(c, c % _SLOTS).wait()

    tail_col = 999_424 + lax.div(wid, _NUM_BANDS) * 128
    stage = bufs.at[0, :, pl.ds(0, 128)]
    pltpu.sync_copy(
        src_hbm.at[pl.ds(row0, _BAND_ROWS), pl.ds(pl.multiple_of(tail_col, 128), 128)],
        stage,
    )
    pltpu.sync_copy(
        stage,
        dst_hbm.at[pl.ds(row0, _BAND_ROWS), pl.ds(pl.multiple_of(tail_col, 128), 128)],
    )


def _tc_tail_body(src_ref, _aliased_ref, out_ref):
    out_ref[...] = src_ref[...]


def kernel(inputs, ent_embeds):
    del inputs
    rows, dim = ent_embeds.shape
    wide = ent_embeds.T
    mesh = plsc.VectorSubcoreMesh(
        core_axis_name="c",
        subcore_axis_name="s",
        num_cores=_NUM_CORES,
        num_subcores=_NUM_SUBCORES,
    )
    sc_copy = pl.kernel(
        _sc_copy_body,
        out_type=jax.ShapeDtypeStruct(wide.shape, wide.dtype),
        mesh=mesh,
        scratch_types=[
            pltpu.VMEM((_SLOTS, _BAND_ROWS, _CHUNK_COLS), jnp.float32),
            pltpu.SemaphoreType.DMA((_SLOTS,)),
            pltpu.SemaphoreType.DMA((_SLOTS,)),
        ],
    )
    sc_out = sc_copy(wide)

    tail_idx = wide.shape[1] // 128
    out = pl.pallas_call(
        _tc_tail_body,
        out_shape=jax.ShapeDtypeStruct(wide.shape, wide.dtype),
        grid=(1,),
        in_specs=[
            pl.BlockSpec((dim, 128), lambda i: (0, tail_idx)),
            pl.BlockSpec(memory_space=pltpu.MemorySpace.HBM),
        ],
        out_specs=pl.BlockSpec((dim, 128), lambda i: (0, tail_idx)),
        input_output_aliases={1: 0},
    )(wide, sc_out)
    return out.T


# speedup vs baseline: 1.2755x; 1.2755x over previous
import jax
import jax.numpy as jnp
from jax.experimental import pallas as pl
from jax.experimental.pallas import tpu as pltpu

_BLOCK_COLS = 49152


def _copy_body(src_ref, dst_ref):
    dst_ref[...] = src_ref[...]


def kernel(inputs, ent_embeds):
    del inputs
    rows, dim = ent_embeds.shape
    wide = ent_embeds.T
    grid = (rows + _BLOCK_COLS - 1) // _BLOCK_COLS
    out = pl.pallas_call(
        _copy_body,
        out_shape=jax.ShapeDtypeStruct(wide.shape, wide.dtype),
        grid=(grid,),
        in_specs=[pl.BlockSpec((dim, _BLOCK_COLS), lambda i: (0, i))],
        out_specs=pl.BlockSpec((dim, _BLOCK_COLS), lambda i: (0, i)),
    )(wide)
    return out.T


# speedup vs baseline: 1.2761x; 1.0005x over previous
import jax
import jax.numpy as jnp
from jax.experimental import pallas as pl
from jax.experimental.pallas import tpu as pltpu

_BLOCK_COLS = 57344


def _copy_body(src_ref, dst_ref):
    dst_ref[...] = src_ref[...]


def kernel(inputs, ent_embeds):
    del inputs
    rows, dim = ent_embeds.shape
    wide = ent_embeds.T
    grid = (rows + _BLOCK_COLS - 1) // _BLOCK_COLS
    out = pl.pallas_call(
        _copy_body,
        out_shape=jax.ShapeDtypeStruct(wide.shape, wide.dtype),
        grid=(grid,),
        in_specs=[pl.BlockSpec((dim, _BLOCK_COLS), lambda i: (0, i))],
        out_specs=pl.BlockSpec((dim, _BLOCK_COLS), lambda i: (0, i)),
    )(wide)
    return out.T
